# baseline (speedup 1.0000x reference)
import functools

import jax
import jax.numpy as jnp
from jax import lax
from jax.experimental import pallas as pl
from jax.experimental.pallas import tpu as pltpu
from jax.experimental.pallas import tpu_sc as plsc

N = 10000
D = 128
E = 320000
NC = 2
NS = 16
NW = NC * NS
EPW = E // NW
K = 40
NCHUNK = EPW // K
NP = 10240
RPT = NP // NS


@functools.partial(
    pl.kernel,
    mesh=plsc.VectorSubcoreMesh(core_axis_name="c", subcore_axis_name="s"),
    out_type=(
        jax.ShapeDtypeStruct((NC, NP, D), jnp.float32),
        jax.ShapeDtypeStruct((NW, NP // D, D), jnp.float32),
    ),
    scratch_types=[
        pltpu.VMEM_SHARED((NP, D), jnp.float32),
        pltpu.VMEM((NCHUNK, K), jnp.int32),
        pltpu.VMEM((NCHUNK, K), jnp.int32),
        pltpu.VMEM((5 * K, D), jnp.float32),
        pltpu.SemaphoreType.DMA,
        pltpu.SemaphoreType.DMA,
        pltpu.SemaphoreType.DMA,
        pltpu.SemaphoreType.DMA,
        pltpu.SemaphoreType.DMA,
        pltpu.SemaphoreType.DMA,
        pltpu.SemaphoreType.DMA,
        pltpu.SemaphoreType.DMA,
        pltpu.SemaphoreType.DMA,
        pltpu.SemaphoreType.DMA,
        pltpu.SemaphoreType.DMA,
        pltpu.SemaphoreType.DMA,
    ],
    compiler_params=pltpu.CompilerParams(use_tc_tiling_on_sc=False,
                                         needs_layout_passes=False),
)
def _sc_aggregate(tab, src, dst, zz, out, out_deg, acc, idx_s, idx_d, rows,
                  g0, g1, g2, g3, g4, s0, s1, s2, s3, s4, z0, z1):
    cid = lax.axis_index("c")
    sid = lax.axis_index("s")
    wid = sid * NC + cid
    sem_g = (g0, g1, g2, g3, g4)
    sem_s = (s0, s1, s2, s3, s4)
    HR = NP // D

    def gath(c, b):
        return pltpu.make_async_copy(tab.at[idx_s.at[c]],
                                     rows.at[pl.ds(b * K, K)], sem_g[b])

    def scat(c, b):
        return pltpu.make_async_copy(rows.at[pl.ds(b * K, K)],
                                     acc.at[idx_d.at[c]], sem_s[b])

    cs = pltpu.make_async_copy(src.at[wid], idx_s, s0)
    cd = pltpu.make_async_copy(dst.at[wid], idx_d, s1)
    zeroacc = pltpu.make_async_copy(zz, acc.at[pl.ds(sid * RPT, RPT)], z0)
    cs.start()
    cd.start()
    zeroacc.start()
    cs.wait()
    gath(0, 0).start()
    gath(1, 1).start()
    gath(2, 2).start()

    hz = pltpu.make_async_copy(zz.at[pl.ds(0, HR)],
                               rows.at[pl.ds(3 * K, HR)], z1)
    hz.start()
    cd.wait()
    hz.wait()
    ones = jnp.ones((16,), jnp.float32)
    tailmask = lax.iota(jnp.int32, 16) >= 8

    def hadd1(v, mask=None):
        plsc.addupdate_scatter(rows, [(v >> 7) + 3 * K, v & 127], ones,
                               mask=mask)

    def hadd(c, carry):
        hadd1(idx_d[c, pl.ds(0, 16)])
        hadd1(idx_d[c, pl.ds(16, 16)])
        hadd1(idx_d[c, pl.ds(24, 16)], tailmask)
        return carry

    lax.fori_loop(0, NCHUNK, hadd, 0)
    pltpu.sync_copy(rows.at[pl.ds(3 * K, HR)], out_deg.at[wid])

    zeroacc.wait()
    plsc.subcore_barrier()

    gath(0, 0).wait()
    scat(0, 0).start(add=True)
    gath(3, 3).start()
    gath(1, 1).wait()
    scat(1, 1).start(add=True)
    gath(4, 4).start()
    for c in (2, 3, 4):
        scat(c - 2, c - 2).wait()
        gath(c + 3, c - 2).start()
        gath(c, c).wait()
        scat(c, c).start(add=True)

    def group(m, carry):
        for j in range(5):
            c = 5 * m + j
            bn = (j + 3) % 5
            scat(c - 2, bn).wait()
            nxt = jnp.minimum(c + 3, NCHUNK - 1)
            gath(nxt, bn).start()
            gath(c, j).wait()
            scat(c, j).start(add=True)
        return carry

    lax.fori_loop(1, NCHUNK // 5, group, 0)
    scat(NCHUNK - 2, 3).wait()
    scat(NCHUNK - 1, 4).wait()
    gath(NCHUNK - 1, 0).wait()
    gath(NCHUNK - 1, 1).wait()
    gath(NCHUNK - 1, 2).wait()
    plsc.subcore_barrier()

    pltpu.sync_copy(acc.at[pl.ds(sid * RPT, RPT)],
                    out.at[cid, pl.ds(sid * RPT, RPT)])


R = 1024
HB = R // D


def _affine_body(p_ref, h_ref, w_ref, b_ref, o_ref, *, relu):
    p = p_ref[0] + p_ref[1]
    deg8 = jnp.maximum(h_ref[...].sum(axis=0), 1.0)
    ri = lax.broadcasted_iota(jnp.int32, (R, HB), 0)
    si = lax.broadcasted_iota(jnp.int32, (R, HB), 1)
    sel = ((ri >> 7) == si).astype(jnp.float32)
    brows = jnp.dot(sel, deg8, preferred_element_type=jnp.float32)
    ii = lax.broadcasted_iota(jnp.int32, (R, D), 0)
    jj = lax.broadcasted_iota(jnp.int32, (R, D), 1)
    deg = jnp.sum(jnp.where((ii & 127) == jj, brows, 0.0), axis=1,
                  keepdims=True)
    a = p / deg
    h = jnp.dot(a, w_ref[...], preferred_element_type=jnp.float32) + b_ref[...]
    if relu:
        h = jnp.maximum(h, 0.0)
    o_ref[...] = h


def _tc_affine(partials, hist, w, b, *, relu):
    return pl.pallas_call(
        functools.partial(_affine_body, relu=relu),
        grid=(NP // R,),
        in_specs=[
            pl.BlockSpec((NC, R, D), lambda i: (0, i, 0)),
            pl.BlockSpec((NW, HB, D), lambda i: (0, i, 0)),
            pl.BlockSpec((D, D), lambda i: (0, 0)),
            pl.BlockSpec((1, D), lambda i: (0, 0)),
        ],
        out_specs=pl.BlockSpec((R, D), lambda i: (i, 0)),
        out_shape=jax.ShapeDtypeStruct((N, D), jnp.float32),
    )(partials, hist, w, b.reshape(1, D))


def kernel(x, edge_index_list, W0, b0, W1, b1):
    zz = jnp.zeros((RPT, D), jnp.float32)
    e = edge_index_list.reshape(2, 2, NW, NCHUNK, K)

    p1, hist1 = _sc_aggregate(x, e[0, 0], e[0, 1], zz)
    h1 = _tc_affine(p1, hist1, W0, b0, relu=True)
    p2, hist2 = _sc_aggregate(h1, e[1, 0], e[1, 1], zz)
    out = _tc_affine(p2, hist2, W1, b1, relu=False)
    return out
